# baseline (speedup 1.0000x reference)
import jax
import jax.numpy as jnp
from jax import lax
from jax.experimental import pallas as pl
from jax.experimental.pallas import tpu as pltpu

N_NODES = 10000
NP = 10240
D_IN = 128
D_OUT = 32
PACK = 40

BRP = 1280
BR = 400
BC = 10240
NEG = -1.0e30


def _prep_body(x_ref, w_ref, b_ref, asc_ref, adc_ref, adr_ref,
               xb2_ref, ec_ref, fc_ref, fr_ref, fg_ref):
    i = pl.program_id(0)
    xb = lax.dot_general(x_ref[...], w_ref[...], (((1,), (1,)), ((), ())),
                         preferred_element_type=jnp.float32) + b_ref[...]
    rid = i * BRP + lax.broadcasted_iota(jnp.int32, (BRP, 1), 0)
    rmask = rid < N_NODES
    xb2 = jnp.concatenate(
        [xb,
         jnp.ones((BRP, 1), jnp.float32),
         jnp.zeros((BRP, PACK - D_OUT - 1), jnp.float32)], axis=1)
    xb2_ref[...] = jnp.where(rmask, xb2, 0.0)
    ec = lax.dot_general(xb, asc_ref[...], (((1,), (0,)), ((), ())),
                         preferred_element_type=jnp.float32)
    fc = lax.dot_general(xb, adc_ref[...], (((1,), (0,)), ((), ())),
                         preferred_element_type=jnp.float32)
    ec_ref[...] = jnp.where(rmask, ec, NEG)
    fc_ref[...] = jnp.where(rmask, fc, NEG)
    tr = lax.dot_general(adr_ref[...], xb, (((1,), (1,)), ((), ())),
                         preferred_element_type=jnp.float32)
    lid = i * BRP + lax.broadcasted_iota(jnp.int32, (1, BRP), 1)
    trm = jnp.where(lid < N_NODES, tr, NEG)
    fr_ref[...] = trm
    bm = jnp.max(trm, axis=1, keepdims=True)

    @pl.when(i == 0)
    def _first():
        fg_ref[...] = bm

    @pl.when(i > 0)
    def _rest():
        fg_ref[...] = jnp.maximum(fg_ref[...], bm)


def _att_body(ec_ref, fc_ref, fr_ref, fg_ref, a_ref, xbj_ref, xbi_ref,
              out_ref, acc_ref):
    j = pl.program_id(1)
    nj = pl.num_programs(1)
    fg = fg_ref[0, 0]
    e = ec_ref[...]
    u = e + fg
    m = jnp.where(u >= 0.0, u, 0.01 * u)
    ebig = jnp.exp(u - m)
    esml = jnp.exp(0.01 * u - m)
    f = fr_ref[...]
    fbig = jnp.exp(f - fg)
    fsml = jnp.exp(0.01 * (f - fg))
    w = jnp.maximum(ebig * fbig, esml * fsml)
    p = jnp.where(a_ref[...] != 0, w, 0.0)
    contrib = lax.dot_general(p, xbj_ref[...], (((1,), (0,)), ((), ())),
                              preferred_element_type=jnp.float32)

    @pl.when(j == 0)
    def _init():
        fi = fc_ref[...]
        s = e + fi
        ls = jnp.where(s >= 0.0, s, 0.01 * s)
        wself = jnp.exp(ls - m)
        acc_ref[...] = wself * xbi_ref[...] + contrib

    @pl.when(j > 0)
    def _acc():
        acc_ref[...] = acc_ref[...] + contrib

    @pl.when(j == nj - 1)
    def _fin():
        accv = acc_ref[...]
        out_ref[...] = accv[:, :D_OUT] / accv[:, D_OUT:D_OUT + 1]


def kernel(X, A, W, b, a):
    asc = a[0, :D_OUT].reshape(D_OUT, 1)
    adc = a[0, D_OUT:].reshape(D_OUT, 1)
    adr = a[0, D_OUT:].reshape(1, D_OUT)
    b2 = b.reshape(1, D_OUT)

    xb2, ecol, fcol, frow, fg = pl.pallas_call(
        _prep_body,
        grid=(NP // BRP,),
        in_specs=[
            pl.BlockSpec((BRP, D_IN), lambda i: (i, 0)),
            pl.BlockSpec((D_OUT, D_IN), lambda i: (0, 0)),
            pl.BlockSpec((1, D_OUT), lambda i: (0, 0)),
            pl.BlockSpec((D_OUT, 1), lambda i: (0, 0)),
            pl.BlockSpec((D_OUT, 1), lambda i: (0, 0)),
            pl.BlockSpec((1, D_OUT), lambda i: (0, 0)),
        ],
        out_specs=[
            pl.BlockSpec((BRP, PACK), lambda i: (i, 0)),
            pl.BlockSpec((BRP, 1), lambda i: (i, 0)),
            pl.BlockSpec((BRP, 1), lambda i: (i, 0)),
            pl.BlockSpec((1, BRP), lambda i: (0, i)),
            pl.BlockSpec((1, 1), lambda i: (0, 0)),
        ],
        out_shape=[
            jax.ShapeDtypeStruct((NP, PACK), jnp.float32),
            jax.ShapeDtypeStruct((NP, 1), jnp.float32),
            jax.ShapeDtypeStruct((NP, 1), jnp.float32),
            jax.ShapeDtypeStruct((1, NP), jnp.float32),
            jax.ShapeDtypeStruct((1, 1), jnp.float32),
        ],
    )(X, W, b2, asc, adc, adr)

    out = pl.pallas_call(
        _att_body,
        grid=(N_NODES // BR, NP // BC),
        in_specs=[
            pl.BlockSpec((BR, 1), lambda i, j: (i, 0)),
            pl.BlockSpec((BR, 1), lambda i, j: (i, 0)),
            pl.BlockSpec((1, BC), lambda i, j: (0, j)),
            pl.BlockSpec((1, 1), lambda i, j: (0, 0)),
            pl.BlockSpec((BR, BC), lambda i, j: (i, j)),
            pl.BlockSpec((BC, PACK), lambda i, j: (j, 0)),
            pl.BlockSpec((BR, PACK), lambda i, j: (i, 0)),
        ],
        out_specs=pl.BlockSpec((BR, D_OUT), lambda i, j: (i, 0)),
        out_shape=jax.ShapeDtypeStruct((N_NODES, D_OUT), jnp.float32),
        scratch_shapes=[pltpu.VMEM((BR, PACK), jnp.float32)],
    )(ecol, fcol, frow, fg, A, xb2, xb2)
    return out


# speedup vs baseline: 1.1100x; 1.1100x over previous
import jax
import jax.numpy as jnp
from jax import lax
from jax.experimental import pallas as pl
from jax.experimental.pallas import tpu as pltpu

N_NODES = 10000
D_IN = 128
D_OUT = 32
PACK = 40

BR = 400


def _att_body(x_ref, w_ref, b_ref, asc_ref, adc_ref, adr_ref, a_ref,
              out_ref, xb2_s, ec_s, fc_s, fr_s, fg_s):
    i = pl.program_id(0)

    @pl.when(i == 0)
    def _prep():
        xb = lax.dot_general(x_ref[...], w_ref[...],
                             (((1,), (1,)), ((), ())),
                             preferred_element_type=jnp.float32) + b_ref[...]
        xb2_s[...] = jnp.zeros((N_NODES, PACK), jnp.float32)
        xb2_s[:, 0:D_OUT] = xb
        xb2_s[:, D_OUT:D_OUT + 1] = jnp.ones((N_NODES, 1), jnp.float32)
        ec_s[...] = lax.dot_general(xb, asc_ref[...],
                                    (((1,), (0,)), ((), ())),
                                    preferred_element_type=jnp.float32)
        fc_s[...] = lax.dot_general(xb, adc_ref[...],
                                    (((1,), (0,)), ((), ())),
                                    preferred_element_type=jnp.float32)
        fr = lax.dot_general(adr_ref[...], xb, (((1,), (1,)), ((), ())),
                             preferred_element_type=jnp.float32)
        fr_s[...] = fr
        fg_s[...] = jnp.max(fr, axis=1, keepdims=True)

    fg = fg_s[0, 0]
    e = ec_s[pl.ds(i * BR, BR), :]
    u = e + fg
    m = jnp.where(u >= 0.0, u, 0.01 * u)
    ebig = jnp.exp(u - m)
    esml = jnp.exp(0.01 * u - m)
    f = fr_s[...]
    fbig = jnp.exp(f - fg)
    fsml = jnp.exp(0.01 * (f - fg))
    w = jnp.maximum(ebig * fbig, esml * fsml)
    p = jnp.where(a_ref[...] != 0, w, 0.0)
    contrib = lax.dot_general(p, xb2_s[...], (((1,), (0,)), ((), ())),
                              preferred_element_type=jnp.float32)

    fi = fc_s[pl.ds(i * BR, BR), :]
    s = e + fi
    ls = jnp.where(s >= 0.0, s, 0.01 * s)
    wself = jnp.exp(ls - m)
    acc = wself * xb2_s[pl.ds(i * BR, BR), :] + contrib
    out_ref[...] = acc[:, :D_OUT] / acc[:, D_OUT:D_OUT + 1]


def kernel(X, A, W, b, a):
    asc = a[0, :D_OUT].reshape(D_OUT, 1)
    adc = a[0, D_OUT:].reshape(D_OUT, 1)
    adr = a[0, D_OUT:].reshape(1, D_OUT)
    b2 = b.reshape(1, D_OUT)

    out = pl.pallas_call(
        _att_body,
        grid=(N_NODES // BR,),
        in_specs=[
            pl.BlockSpec((N_NODES, D_IN), lambda i: (0, 0)),
            pl.BlockSpec((D_OUT, D_IN), lambda i: (0, 0)),
            pl.BlockSpec((1, D_OUT), lambda i: (0, 0)),
            pl.BlockSpec((D_OUT, 1), lambda i: (0, 0)),
            pl.BlockSpec((D_OUT, 1), lambda i: (0, 0)),
            pl.BlockSpec((1, D_OUT), lambda i: (0, 0)),
            pl.BlockSpec((BR, N_NODES), lambda i: (i, 0)),
        ],
        out_specs=pl.BlockSpec((BR, D_OUT), lambda i: (i, 0)),
        out_shape=jax.ShapeDtypeStruct((N_NODES, D_OUT), jnp.float32),
        scratch_shapes=[
            pltpu.VMEM((N_NODES, PACK), jnp.float32),
            pltpu.VMEM((N_NODES, 1), jnp.float32),
            pltpu.VMEM((N_NODES, 1), jnp.float32),
            pltpu.VMEM((1, N_NODES), jnp.float32),
            pltpu.VMEM((1, 1), jnp.float32),
        ],
    )(X, W, b2, asc, adc, adr, A)
    return out
